# Initial kernel scaffold; baseline (speedup 1.0000x reference)
import jax
import jax.numpy as jnp
from jax.experimental import pallas as pl


def kernel(a_hat, in_feat, w1, b1, w2, b2):
    raise NotImplementedError("write your pallas kernel here")


# speedup vs baseline: 1.5777x; 1.5777x over previous
import functools

import jax
import jax.numpy as jnp
from jax.experimental import pallas as pl
from jax.experimental.pallas import tpu as pltpu


def _vmem_limit_bytes():
    return 48 * 1024 * 1024


def _xw1_kernel(x_ref, w1_ref, p_ref):
    p_ref[...] = jnp.dot(
        x_ref[...].astype(jnp.bfloat16), w1_ref[...],
        preferred_element_type=jnp.float32,
    ).astype(p_ref.dtype)


def _layer1_kernel(a_ref, p_ref, b1_ref, w2_ref, q_ref):
    ap = jnp.dot(a_ref[...].astype(jnp.bfloat16), p_ref[...],
                 preferred_element_type=jnp.float32)
    h = jnp.maximum(ap + b1_ref[...], 0.0)
    q_ref[...] = jnp.dot(h.astype(jnp.bfloat16), w2_ref[...],
                         preferred_element_type=jnp.float32).astype(q_ref.dtype)


def _layer2_kernel(a_ref, q_ref, b2_ref, out_ref):
    aq = jnp.dot(a_ref[...].astype(jnp.bfloat16), q_ref[...],
                 preferred_element_type=jnp.float32)
    out_ref[...] = jnp.maximum(aq + b2_ref[...], 0.0)


@jax.jit
def _gcn(a_hat, in_feat, w1, b1, w2, b2):
    n, f = in_feat.shape
    h = w1.shape[1]
    c = w2.shape[1]

    w1_b = w1.astype(jnp.bfloat16)
    w2_b = w2.astype(jnp.bfloat16)
    b1_f = b1.astype(jnp.float32).reshape(1, h)
    b2_f = b2.astype(jnp.float32).reshape(1, c)

    cparams = pltpu.CompilerParams(
        dimension_semantics=("parallel",),
        vmem_limit_bytes=_vmem_limit_bytes(),
    )

    tp = 1024
    p = pl.pallas_call(
        _xw1_kernel,
        grid=(n // tp,),
        out_shape=jax.ShapeDtypeStruct((n, h), jnp.bfloat16),
        in_specs=[
            pl.BlockSpec((tp, f), lambda i: (i, 0)),
            pl.BlockSpec((f, h), lambda i: (0, 0)),
        ],
        out_specs=pl.BlockSpec((tp, h), lambda i: (i, 0)),
        compiler_params=cparams,
        cost_estimate=pl.CostEstimate(
            flops=2 * n * f * h, transcendentals=0,
            bytes_accessed=4 * n * f + 2 * (f * h + n * h)),
    )(in_feat, w1_b)

    tm = 256
    l1_bytes = 4 * n * n + 2 * (n * h + h * c + n * c) + 4 * h
    q = pl.pallas_call(
        _layer1_kernel,
        grid=(n // tm,),
        out_shape=jax.ShapeDtypeStruct((n, c), jnp.bfloat16),
        in_specs=[
            pl.BlockSpec((tm, n), lambda i: (i, 0)),
            pl.BlockSpec((n, h), lambda i: (0, 0)),
            pl.BlockSpec((1, h), lambda i: (0, 0)),
            pl.BlockSpec((h, c), lambda i: (0, 0)),
        ],
        out_specs=pl.BlockSpec((tm, c), lambda i: (i, 0)),
        compiler_params=cparams,
        cost_estimate=pl.CostEstimate(
            flops=2 * n * n * h + 2 * n * h * c, transcendentals=0,
            bytes_accessed=l1_bytes),
    )(a_hat, p, b1_f, w2_b)

    l2_bytes = 4 * n * n + 2 * n * c + 4 * (c + n * c)
    out = pl.pallas_call(
        _layer2_kernel,
        grid=(n // tm,),
        out_shape=jax.ShapeDtypeStruct((n, c), jnp.float32),
        in_specs=[
            pl.BlockSpec((tm, n), lambda i: (i, 0)),
            pl.BlockSpec((n, c), lambda i: (0, 0)),
            pl.BlockSpec((1, c), lambda i: (0, 0)),
        ],
        out_specs=pl.BlockSpec((tm, c), lambda i: (i, 0)),
        compiler_params=cparams,
        cost_estimate=pl.CostEstimate(
            flops=2 * n * n * c, transcendentals=0,
            bytes_accessed=l2_bytes),
    )(a_hat, q, b2_f)

    return out


def kernel(a_hat, in_feat, w1, b1, w2, b2):
    return _gcn(a_hat, in_feat, w1, b1, w2, b2)
